# baseline (speedup 1.0000x reference)
import functools

import jax
import jax.numpy as jnp
from jax import lax
from jax.experimental import pallas as pl
from jax.experimental.pallas import tpu as pltpu
from jax.experimental.pallas import tpu_sc as plsc

B = 4096
L = 10
T = 20
N = B * L
D_AB = 16
D_T = 32
D_OUT = 64

NC = 2
NS = 16
NW = NC * NS
PER_TILE = N // NW
CH = 128
NCHUNK = PER_TILE // CH
SUB = 128
NSUB = CH * T // SUB
SCALE = 1.0 / T


PREP_R = 624
PREP_ROUNDS = [(0, 256), (256, 256), (512, 112)]
TAIL = NS * PREP_R


def _sc_body(ida, idb, idt, e_id, ec1_t, et_t, out, tex_c, c1_c,
             idxa0, idxa1, idxb0, idxb1, idxt0, idxt1,
             av0, av1, bv0, bv1, tv0, tv1, ov0, ov1,
             stage, res_bf, res_f,
             si0, si1, sg0, sg1, so0, so1):
    idxa = [idxa0, idxa1]
    idxb = [idxb0, idxb1]
    idxt = [idxt0, idxt1]
    av = [av0, av1]
    bv = [bv0, bv1]
    tv = [tv0, tv1]
    ov = [ov0, ov1]
    si = [si0, si1]
    sg = [sg0, sg1]
    so = [so0, so1]

    cid = lax.axis_index("c")
    tid = lax.axis_index("s")
    wid = tid * NC + cid
    base = wid * PER_TILE
    e_text = tex_c.at[cid]
    e_c1 = c1_c.at[cid]

    def idx_copies(gg, s):
        n0 = base + gg * CH
        l = n0 // B
        b0 = n0 - l * B
        return [
            pltpu.make_async_copy(ida.at[pl.ds(n0, CH)], idxa[s], si[s]),
            pltpu.make_async_copy(idb.at[pl.ds(n0, CH)], idxb[s], si[s]),
            pltpu.make_async_copy(idt.at[pl.ds(l * T, T), pl.ds(b0, CH)],
                                  idxt[s], si[s]),
        ]

    def gather_copies(s):
        cps = [
            pltpu.make_async_copy(e_id.at[idxa[s]], av[s], sg[s]),
            pltpu.make_async_copy(e_c1.at[idxb[s]], bv[s], sg[s]),
        ]
        for j in range(T):
            cps.append(pltpu.make_async_copy(
                e_text.at[idxt[s].at[j]],
                tv[s].at[pl.ds(j * CH, CH)], sg[s]))
        return cps

    def out_copies(gg, s):
        n0 = base + gg * CH
        return [
            pltpu.make_async_copy(av[s], out.at[pl.ds(n0, CH), pl.ds(0, 16)],
                                  so[s]),
            pltpu.make_async_copy(bv[s], out.at[pl.ds(n0, CH), pl.ds(16, 16)],
                                  so[s]),
            pltpu.make_async_copy(ov[s], out.at[pl.ds(n0, CH), pl.ds(32, 32)],
                                  so[s]),
        ]

    def wait_out(s):
        for c in out_copies(0, s):
            c.wait()

    def fire_idx(gg, s):
        for c in idx_copies(gg, s):
            c.start()

    def wait_idx(s):
        for c in idx_copies(0, s):
            c.wait()

    def fire_gathers(s):
        for c in gather_copies(s):
            c.start()

    def wait_gathers(s):
        for c in gather_copies(s):
            c.wait()

    def compute(gg, s):
        t_v, o_v = tv[s], ov[s]

        @plsc.parallel_loop(0, CH, unroll=2)
        def item_body(i):
            acc = t_v[i, :]
            for j in range(1, T):
                acc = acc + t_v[j * CH + i, :]
            acc_a, acc_b = plsc.unpack(
                acc, format=plsc.PackFormat.INTERLEAVED)
            o_v[i, pl.ds(0, 16)] = acc_a * SCALE
            o_v[i, pl.ds(16, 16)] = acc_b * SCALE

        for c in out_copies(gg, s):
            c.start()

    fire_idx(0, 0)
    fire_idx(1, 1)

    rows16 = jax.lax.iota(jnp.int32, 16)
    tbase = pl.multiple_of(tid * PREP_R, 8)

    def conv_text_round(start, sz):
        pltpu.sync_copy(et_t.at[:, pl.ds(start, sz)],
                        stage.at[:, pl.ds(0, sz)])

        @plsc.parallel_loop(0, sz, unroll=2)
        def conv_t(r):
            col = jnp.full((16,), r, jnp.int32)
            lo = plsc.load_gather(stage, [rows16, col])
            hi = plsc.load_gather(stage, [rows16 + 16, col])
            res_bf[r, :] = plsc.pack(lo, hi,
                                     format=plsc.PackFormat.INTERLEAVED)

        pltpu.sync_copy(res_bf.at[pl.ds(0, sz)],
                        tex_c.at[cid, pl.ds(start, sz)])

    def conv_c1_round(start, sz):
        pltpu.sync_copy(ec1_t.at[:, pl.ds(start, sz)],
                        stage.at[pl.ds(0, 16), pl.ds(0, sz)])

        @plsc.parallel_loop(0, sz, unroll=2)
        def conv_c(r):
            col = jnp.full((16,), r, jnp.int32)
            res_f[r, :] = plsc.load_gather(stage, [rows16, col])

        pltpu.sync_copy(res_f.at[pl.ds(0, sz)],
                        c1_c.at[cid, pl.ds(start, sz)])

    for off, sz in PREP_ROUNDS:
        conv_text_round(tbase + off, sz)
    for off, sz in PREP_ROUNDS:
        conv_c1_round(tbase + off, sz)

    @pl.when(tid == NS - 1)
    def _tail():
        conv_text_round(TAIL, 10000 - TAIL)
        conv_c1_round(TAIL, 10008 - TAIL)

    plsc.subcore_barrier()

    wait_idx(0)
    fire_gathers(0)

    def loop_body(gi, carry):
        for s in (0, 1):
            gg = 2 * gi + s
            wait_gathers(s)
            fire_idx(gg + 2, s)
            wait_idx(1 - s)

            @pl.when(gg >= 1)
            def _():
                wait_out(1 - s)

            fire_gathers(1 - s)
            compute(gg, s)
        return carry

    lax.fori_loop(0, (NCHUNK - 2) // 2, loop_body, 0)

    wait_gathers(0)
    wait_idx(1)
    wait_out(1)
    fire_gathers(1)
    compute(NCHUNK - 2, 0)
    wait_gathers(1)
    compute(NCHUNK - 1, 1)
    wait_out(0)
    wait_out(1)


_sc_call = functools.partial(
    pl.kernel,
    mesh=plsc.VectorSubcoreMesh(core_axis_name="c", subcore_axis_name="s"),
    out_type=(
        jax.ShapeDtypeStruct((N, D_OUT), jnp.float32),
        jax.ShapeDtypeStruct((NC, 10000, D_T), jnp.bfloat16),
        jax.ShapeDtypeStruct((NC, 10008, D_AB), jnp.float32),
    ),
    compiler_params=pltpu.CompilerParams(use_tc_tiling_on_sc=False,
                                         needs_layout_passes=False),
    scratch_types=[
        pltpu.VMEM((CH,), jnp.int32),
        pltpu.VMEM((CH,), jnp.int32),
        pltpu.VMEM((CH,), jnp.int32),
        pltpu.VMEM((CH,), jnp.int32),
        pltpu.VMEM((T, CH), jnp.int32),
        pltpu.VMEM((T, CH), jnp.int32),
        pltpu.VMEM((CH, D_AB), jnp.float32),
        pltpu.VMEM((CH, D_AB), jnp.float32),
        pltpu.VMEM((CH, D_AB), jnp.float32),
        pltpu.VMEM((CH, D_AB), jnp.float32),
        pltpu.VMEM((CH * T, D_T), jnp.bfloat16),
        pltpu.VMEM((CH * T, D_T), jnp.bfloat16),
        pltpu.VMEM((CH, D_T), jnp.float32),
        pltpu.VMEM((CH, D_T), jnp.float32),
        pltpu.VMEM((32, 256), jnp.float32),
        pltpu.VMEM((256, D_T), jnp.bfloat16),
        pltpu.VMEM((256, D_AB), jnp.float32),
        pltpu.SemaphoreType.DMA,
        pltpu.SemaphoreType.DMA,
        pltpu.SemaphoreType.DMA,
        pltpu.SemaphoreType.DMA,
        pltpu.SemaphoreType.DMA,
        pltpu.SemaphoreType.DMA,
    ],
)(_sc_body)


def kernel(item_id, contex_1, contex_2_tokens, E_id, E_c1, E_text):
    ida = jnp.transpose(item_id).reshape(N).astype(jnp.int32)
    idb = jnp.transpose(contex_1).reshape(N).astype(jnp.int32)
    idt = jnp.transpose(contex_2_tokens, (1, 2, 0)).reshape(L * T, B)
    idt = idt.astype(jnp.int32)
    et_t = jnp.transpose(E_text)
    ec1_t = jnp.pad(jnp.transpose(E_c1), ((0, 0), (0, 7)))
    out, _, _ = _sc_call(ida, idb, idt, E_id, ec1_t, et_t)
    return jnp.transpose(out.reshape(L, B, D_OUT), (1, 0, 2))
